# baseline (speedup 1.0000x reference)
import functools

import jax
import jax.numpy as jnp
from jax import lax
from jax.experimental import pallas as pl
from jax.experimental.pallas import tpu as pltpu
from jax.experimental.pallas import tpu_sc as plsc

NC = 2
NS = 16
NW = NC * NS
LANES = 16


@functools.lru_cache(maxsize=None)
def _make_emb_gather(V, D, B, CH=80):
  bpw = B // NW
  nch = bpw // CH
  assert bpw % CH == 0 and B % NW == 0 and CH % 8 == 0
  mesh = plsc.VectorSubcoreMesh(core_axis_name="c", subcore_axis_name="s", num_cores=NC, num_subcores=NS)

  @functools.partial(
      pl.kernel, mesh=mesh,
      out_type=jax.ShapeDtypeStruct((B, D), jnp.float32),
      scratch_types=[
          pltpu.VMEM((nch, CH), jnp.int32),
          pltpu.VMEM((2, CH, D), jnp.float32),
          pltpu.SemaphoreType.DMA,
          pltpu.SemaphoreType.DMA,
      ],
  )
  def k(table_hbm, idx_hbm, out_hbm, idx_v, rows_v, sem0, sem1):
    wid = lax.axis_index("s") * NC + lax.axis_index("c")
    base = wid * bpw
    pltpu.sync_copy(idx_hbm.at[wid], idx_v)
    sems = (sem0, sem1)
    descs = [None] * nch
    for c in range(nch):
      descs[c] = pltpu.async_copy(
          table_hbm.at[idx_v.at[c]], rows_v.at[c % 2], sems[c % 2])
      if c > 0:
        descs[c - 1].wait()
        pltpu.sync_copy(rows_v.at[(c - 1) % 2],
                        out_hbm.at[pl.ds(base + (c - 1) * CH, CH)])
    descs[nch - 1].wait()
    pltpu.sync_copy(rows_v.at[(nch - 1) % 2],
                    out_hbm.at[pl.ds(base + (nch - 1) * CH, CH)])

  return k


@functools.lru_cache(maxsize=None)
def _make_edge_agg(Npad, D, G0, G1, C=128, GC=5):
  KD = D // LANES
  gmax = max(G0, G1)
  rpt = Npad // NS
  assert Npad % NS == 0 and rpt % C == 0
  mesh = plsc.VectorSubcoreMesh(core_axis_name="c", subcore_axis_name="s", num_cores=NC, num_subcores=NS)

  @functools.partial(
      pl.kernel, mesh=mesh,
      out_type=jax.ShapeDtypeStruct((NC, Npad, D), jnp.float32),
      name="edge_agg",
      scratch_types=[
          pltpu.VMEM((2, GC, C), jnp.int32),
          pltpu.VMEM((2, GC, C), jnp.int32),
          pltpu.VMEM((2, GC, C), jnp.float32),
          pltpu.VMEM((2, C, D), jnp.float32),
          pltpu.VMEM_SHARED((Npad, D), jnp.float32),
          pltpu.SemaphoreType.DMA,
          pltpu.SemaphoreType.DMA,
          pltpu.SemaphoreType.DMA,
          pltpu.SemaphoreType.DMA,
      ],
  )
  def k(h_hbm, src_hbm, dst_hbm, w_hbm, out_hbm,
        src_v, dst_v, w_v, rows_v, agg_sh, sem0, sem1, semi, sems):
    cid = lax.axis_index("c")
    sid = lax.axis_index("s")
    gsems = (sem0, sem1)
    ngrp = jnp.where(cid == 0, G0, G1)

    zero = jnp.zeros((LANES,), jnp.float32)
    def zrow(i, _):
      for k8 in range(KD):
        rows_v[0, i, pl.ds(k8 * LANES, LANES)] = zero
      return 0
    lax.fori_loop(0, C, zrow, 0)
    for t in range(rpt // C):
      pltpu.sync_copy(rows_v.at[0], agg_sh.at[pl.ds(sid * rpt + t * C, C)])
    plsc.subcore_barrier()

    def idx_copy(g, par):
      pltpu.async_copy(src_hbm.at[cid, sid, g], src_v.at[par], semi)
      pltpu.async_copy(dst_hbm.at[cid, sid, g], dst_v.at[par], semi)
      pltpu.async_copy(w_hbm.at[cid, sid, g], w_v.at[par], semi)

    def idx_wait(g, par):
      pltpu.make_async_copy(src_hbm.at[cid, sid, g], src_v.at[par], semi).wait()
      pltpu.make_async_copy(dst_hbm.at[cid, sid, g], dst_v.at[par], semi).wait()
      pltpu.make_async_copy(w_hbm.at[cid, sid, g], w_v.at[par], semi).wait()

    def scale_rows(par, jj, buf):
      def grp_body(g16, _):
        w16 = w_v[par, jj, pl.ds(g16 * LANES, LANES)]
        for ei in range(LANES):
          wv = w16[ei]
          e = g16 * LANES + ei
          for k8 in range(KD):
            sl = pl.ds(k8 * LANES, LANES)
            rows_v[buf, e, sl] = rows_v[buf, e, sl] * wv
        return 0
      lax.fori_loop(0, C // LANES, grp_body, 0)

    idx_copy(0, 0)

    def drain_scatter(par, jj, buf):
      pltpu.make_async_copy(
          rows_v.at[buf], agg_sh.at[dst_v.at[par, jj]], sems).wait()

    def group_body(gi, _):
      par = lax.rem(gi, 2)
      idx_wait(gi, par)

      @pl.when(gi + 1 < ngrp)
      def _():
        idx_copy(gi + 1, 1 - par)

      @pl.when(gi > 0)
      def _():
        drain_scatter(par, 0, 0)

      descs = [None] * GC
      descs[0] = pltpu.async_copy(
          h_hbm.at[src_v.at[par, 0]], rows_v.at[0], gsems[0])
      for jj in range(GC):
        descs[jj].wait()
        if jj + 1 < GC:
          if jj == 0:
            @pl.when(gi > 0)
            def _():
              drain_scatter(par, jj, (jj + 1) % 2)
          else:
            drain_scatter(par, jj, (jj + 1) % 2)
          descs[jj + 1] = pltpu.async_copy(
              h_hbm.at[src_v.at[par, jj + 1]], rows_v.at[(jj + 1) % 2],
              gsems[(jj + 1) % 2])
        scale_rows(par, jj, jj % 2)
        pltpu.async_copy(rows_v.at[jj % 2], agg_sh.at[dst_v.at[par, jj]],
                         sems, add=True)
      return 0
    lax.fori_loop(0, ngrp, group_body, 0)

    drain_scatter(lax.rem(ngrp - 1, 2), GC - 1, (GC - 1) % 2)
    drain_scatter(lax.rem(ngrp - 1, 2), GC - 2, (GC - 2) % 2)

    plsc.subcore_barrier()
    pltpu.sync_copy(agg_sh.at[pl.ds(sid * rpt, rpt)],
                    out_hbm.at[cid, pl.ds(sid * rpt, rpt)])

  return k


def _ln_body(h_ref, g_ref, b_ref, o_ref):
  x = h_ref[...]
  mu = jnp.mean(x, axis=1, keepdims=True)
  xc = x - mu
  var = jnp.mean(xc * xc, axis=1, keepdims=True)
  o_ref[...] = xc * lax.rsqrt(var + 1e-5) * g_ref[...] + b_ref[...]


@functools.lru_cache(maxsize=None)
def _make_ln(Npad, D, BLK=1024):
  grid = Npad // BLK
  return pl.pallas_call(
      _ln_body,
      grid=(grid,),
      in_specs=[
          pl.BlockSpec((BLK, D), lambda i: (i, 0)),
          pl.BlockSpec((1, D), lambda i: (0, 0)),
          pl.BlockSpec((1, D), lambda i: (0, 0)),
      ],
      out_specs=pl.BlockSpec((BLK, D), lambda i: (i, 0)),
      out_shape=jax.ShapeDtypeStruct((Npad, D), jnp.float32),
  )


def _conv_mlp_ln_body(hn_ref, p0_ref, p1_ref, W1_ref, b1_ref, W2_ref, b2_ref,
                      eps_ref, g_ref, bb_ref, o_ref):
  hn = hn_ref[...]
  t = (2.0 + eps_ref[0]) * hn + p0_ref[...] + p1_ref[...]
  t = jnp.maximum(t @ W1_ref[...] + b1_ref[...], 0.0) @ W2_ref[...] + b2_ref[...]
  t = jnp.maximum(t, 0.0)
  mu = jnp.mean(t, axis=1, keepdims=True)
  tc = t - mu
  var = jnp.mean(tc * tc, axis=1, keepdims=True)
  o_ref[...] = tc * lax.rsqrt(var + 1e-5) * g_ref[...] + bb_ref[...]


@functools.lru_cache(maxsize=None)
def _make_conv_mlp_ln(Npad, D, H, BLK=512):
  grid = Npad // BLK
  return pl.pallas_call(
      _conv_mlp_ln_body,
      grid=(grid,),
      in_specs=[
          pl.BlockSpec((BLK, D), lambda i: (i, 0)),
          pl.BlockSpec((BLK, D), lambda i: (i, 0)),
          pl.BlockSpec((BLK, D), lambda i: (i, 0)),
          pl.BlockSpec((D, H), lambda i: (0, 0)),
          pl.BlockSpec((1, H), lambda i: (0, 0)),
          pl.BlockSpec((H, H), lambda i: (0, 0)),
          pl.BlockSpec((1, H), lambda i: (0, 0)),
          pl.BlockSpec(memory_space=pltpu.SMEM),
          pl.BlockSpec((1, H), lambda i: (0, 0)),
          pl.BlockSpec((1, H), lambda i: (0, 0)),
      ],
      out_specs=pl.BlockSpec((BLK, H), lambda i: (i, 0)),
      out_shape=jax.ShapeDtypeStruct((Npad, H), jnp.float32),
  )


def _final_body(hn_ref, p0_ref, p1_ref, W1_ref, b1_ref, W2_ref, b2_ref,
                eps_ref, batch_ref, mW1_ref, mb1_ref, mW2_ref, mb2_ref,
                o_ref, sums_ref, cnt_ref, G, BLK):
  i = pl.program_id(0)

  @pl.when(i == 0)
  def _():
    sums_ref[...] = jnp.zeros_like(sums_ref)
    cnt_ref[...] = jnp.zeros_like(cnt_ref)

  hn = hn_ref[...]
  t = (2.0 + eps_ref[0]) * hn + p0_ref[...] + p1_ref[...]
  t = jnp.maximum(t @ W1_ref[...] + b1_ref[...], 0.0) @ W2_ref[...] + b2_ref[...]
  h2 = jnp.maximum(t, 0.0)

  b = batch_ref[0, 0]
  onehot = (b[None, :] == lax.broadcasted_iota(jnp.int32, (G, BLK), 0)
            ).astype(jnp.float32)
  sums_ref[...] += onehot @ h2
  cnt_ref[...] += jnp.broadcast_to(
      jnp.sum(onehot, axis=1, keepdims=True), cnt_ref.shape)

  @pl.when(i == pl.num_programs(0) - 1)
  def _():
    g = sums_ref[...] / jnp.maximum(cnt_ref[...], 1.0)
    o = jnp.maximum(g @ mW1_ref[...] + mb1_ref[...], 0.0) @ mW2_ref[...] \
        + mb2_ref[...]
    o_ref[...] = o


@functools.lru_cache(maxsize=None)
def _make_final(Npad, H, G, BLK=512):
  grid = Npad // BLK
  P = 128
  return pl.pallas_call(
      functools.partial(_final_body, G=G, BLK=BLK),
      grid=(grid,),
      in_specs=[
          pl.BlockSpec((BLK, H), lambda i: (i, 0)),
          pl.BlockSpec((BLK, H), lambda i: (i, 0)),
          pl.BlockSpec((BLK, H), lambda i: (i, 0)),
          pl.BlockSpec((H, H), lambda i: (0, 0)),
          pl.BlockSpec((1, H), lambda i: (0, 0)),
          pl.BlockSpec((H, H), lambda i: (0, 0)),
          pl.BlockSpec((1, H), lambda i: (0, 0)),
          pl.BlockSpec(memory_space=pltpu.SMEM),
          pl.BlockSpec((1, 1, BLK), lambda i: (i, 0, 0)),
          pl.BlockSpec((H, P), lambda i: (0, 0)),
          pl.BlockSpec((1, P), lambda i: (0, 0)),
          pl.BlockSpec((P, P), lambda i: (0, 0)),
          pl.BlockSpec((1, P), lambda i: (0, 0)),
      ],
      out_specs=pl.BlockSpec((G, P), lambda i: (0, 0)),
      out_shape=jax.ShapeDtypeStruct((G, P), jnp.float32),
      scratch_shapes=[
          pltpu.VMEM((G, H), jnp.float32),
          pltpu.VMEM((G, H), jnp.float32),
      ],
  )


def kernel(x, edge_index, edge_attr, batch, emb, ln1_g, ln1_b, W1a, b1a, W2a,
           b2a, eps1, ln2_g, ln2_b, W1b, b1b, W2b, b2b, eps2, mW1, mb1, mW2,
           mb2):
  N = x.shape[0]
  E = edge_index.shape[1]
  V, D = emb.shape
  H = W1a.shape[1]
  G = 64
  OP = mW2.shape[1]
  HH = mW1.shape[1]
  KD = D // LANES
  C = 128

  GC = 5
  Npad = ((N + 8 * NW - 1) // (8 * NW)) * (8 * NW)
  gsz = NS * GC * C
  G0, G1 = 31, 1
  ET = (G0 + G1) * gsz
  assert ET >= E

  x_pad = jnp.zeros((Npad,), jnp.int32).at[:N].set(x.astype(jnp.int32))
  x_pad = x_pad.reshape(NW, -1, 80)

  def _split(a, fill):
    ap = jnp.full((ET,), fill, a.dtype).at[:E].set(a)
    gmax = max(G0, G1)
    p0 = ap[:G0 * gsz].reshape(NS, G0, GC, C)
    p1 = ap[G0 * gsz:].reshape(NS, G1, GC, C)
    p0 = jnp.pad(p0, ((0, 0), (0, gmax - G0), (0, 0), (0, 0)))
    p1 = jnp.pad(p1, ((0, 0), (0, gmax - G1), (0, 0), (0, 0)))
    return jnp.stack([p0, p1])

  src = _split(edge_index[0].astype(jnp.int32), 0)
  dst = _split(edge_index[1].astype(jnp.int32), 0)
  w = _split(edge_attr, 0.0)

  batch_pad = jnp.full((Npad,), G, jnp.int32).at[:N].set(batch.astype(jnp.int32))
  batch_pad = batch_pad.reshape(-1, 1, 512)

  r1 = lambda a: a.reshape(1, -1)

  h = _make_emb_gather(V, D, Npad)(emb, x_pad)
  h1n = _make_ln(Npad, D)(h, r1(ln1_g), r1(ln1_b))

  agg_fn = _make_edge_agg(Npad, D, G0, G1, C)
  parts1 = agg_fn(h1n, src, dst, w)

  h2n = _make_conv_mlp_ln(Npad, D, H)(
      h1n, parts1[0], parts1[1], W1a, r1(b1a), W2a, r1(b2a),
      eps1.reshape(1), r1(ln2_g), r1(ln2_b))

  parts2 = agg_fn(h2n, src, dst, w)

  P = 128
  mW1p = jnp.zeros((H, P), jnp.float32).at[:, :HH].set(mW1)
  mb1p = jnp.zeros((P,), jnp.float32).at[:HH].set(mb1)
  mW2p = jnp.zeros((P, P), jnp.float32).at[:HH, :OP].set(mW2)
  mb2p = jnp.zeros((P,), jnp.float32).at[:OP].set(mb2)

  out = _make_final(Npad, H, G)(
      h2n, parts2[0], parts2[1], W1b, r1(b1b), W2b, r1(b2b),
      eps2.reshape(1), batch_pad, mW1p, r1(mb1p), mW2p, r1(mb2p))
  return out[:, :OP]


# speedup vs baseline: 1.1683x; 1.1683x over previous
import functools

import jax
import jax.numpy as jnp
from jax import lax
from jax.experimental import pallas as pl
from jax.experimental.pallas import tpu as pltpu
from jax.experimental.pallas import tpu_sc as plsc

NC = 2
NS = 16
NW = NC * NS
LANES = 16


@functools.lru_cache(maxsize=None)
def _make_emb_gather(V, D, B, CH=80):
  bpw = B // NW
  nch = bpw // CH
  assert bpw % CH == 0 and B % NW == 0 and CH % 8 == 0
  mesh = plsc.VectorSubcoreMesh(core_axis_name="c", subcore_axis_name="s", num_cores=NC, num_subcores=NS)

  @functools.partial(
      pl.kernel, mesh=mesh,
      out_type=jax.ShapeDtypeStruct((B, D), jnp.float32),
      scratch_types=[
          pltpu.VMEM((nch, CH), jnp.int32),
          pltpu.VMEM((2, CH, D), jnp.float32),
          pltpu.SemaphoreType.DMA,
          pltpu.SemaphoreType.DMA,
      ],
  )
  def k(table_hbm, idx_hbm, out_hbm, idx_v, rows_v, sem0, sem1):
    wid = lax.axis_index("s") * NC + lax.axis_index("c")
    base = wid * bpw
    pltpu.sync_copy(idx_hbm.at[wid], idx_v)
    sems = (sem0, sem1)
    descs = [None] * nch
    for c in range(nch):
      descs[c] = pltpu.async_copy(
          table_hbm.at[idx_v.at[c]], rows_v.at[c % 2], sems[c % 2])
      if c > 0:
        descs[c - 1].wait()
        pltpu.sync_copy(rows_v.at[(c - 1) % 2],
                        out_hbm.at[pl.ds(base + (c - 1) * CH, CH)])
    descs[nch - 1].wait()
    pltpu.sync_copy(rows_v.at[(nch - 1) % 2],
                    out_hbm.at[pl.ds(base + (nch - 1) * CH, CH)])

  return k


@functools.lru_cache(maxsize=None)
def _make_edge_agg(Npad, D, G0, G1, C=128, GC=5):
  KD = D // LANES
  gmax = max(G0, G1)
  rpt = Npad // NS
  assert Npad % NS == 0 and rpt % C == 0
  mesh = plsc.VectorSubcoreMesh(core_axis_name="c", subcore_axis_name="s", num_cores=NC, num_subcores=NS)

  @functools.partial(
      pl.kernel, mesh=mesh,
      out_type=jax.ShapeDtypeStruct((NC, Npad, D), jnp.float32),
      name="edge_agg",
      scratch_types=[
          pltpu.VMEM((2, GC, C), jnp.int32),
          pltpu.VMEM((2, GC, C), jnp.int32),
          pltpu.VMEM((2, GC, C), jnp.float32),
          pltpu.VMEM((2, C, D), jnp.float32),
          pltpu.VMEM_SHARED((Npad, D), jnp.float32),
          pltpu.SemaphoreType.DMA,
          pltpu.SemaphoreType.DMA,
          pltpu.SemaphoreType.DMA,
          pltpu.SemaphoreType.DMA,
      ],
  )
  def k(h_hbm, src_hbm, dst_hbm, w_hbm, out_hbm,
        src_v, dst_v, w_v, rows_v, agg_sh, sem0, sem1, semi, sems):
    cid = lax.axis_index("c")
    sid = lax.axis_index("s")
    gsems = (sem0, sem1)
    ngrp = jnp.where(cid == 0, G0, G1)

    zero = jnp.zeros((LANES,), jnp.float32)
    def zrow(i, _):
      for k8 in range(KD):
        rows_v[0, i, pl.ds(k8 * LANES, LANES)] = zero
      return 0
    lax.fori_loop(0, C, zrow, 0)
    for t in range(rpt // C):
      pltpu.sync_copy(rows_v.at[0], agg_sh.at[pl.ds(sid * rpt + t * C, C)])
    plsc.subcore_barrier()

    def idx_copy(g, par):
      pltpu.async_copy(src_hbm.at[cid, sid, g], src_v.at[par], semi)
      pltpu.async_copy(dst_hbm.at[cid, sid, g], dst_v.at[par], semi)
      pltpu.async_copy(w_hbm.at[cid, sid, g], w_v.at[par], semi)

    def idx_wait(g, par):
      pltpu.make_async_copy(src_hbm.at[cid, sid, g], src_v.at[par], semi).wait()
      pltpu.make_async_copy(dst_hbm.at[cid, sid, g], dst_v.at[par], semi).wait()
      pltpu.make_async_copy(w_hbm.at[cid, sid, g], w_v.at[par], semi).wait()

    def scale_rows(par, jj, buf):
      def grp_body(g16, _):
        w16 = w_v[par, jj, pl.ds(g16 * LANES, LANES)]
        for ei in range(LANES):
          wv = w16[ei]
          e = g16 * LANES + ei
          for k8 in range(KD):
            sl = pl.ds(k8 * LANES, LANES)
            rows_v[buf, e, sl] = rows_v[buf, e, sl] * wv
        return 0
      lax.fori_loop(0, C // LANES, grp_body, 0)

    idx_copy(0, 0)

    def drain_scatter(par, jj, buf):
      pltpu.make_async_copy(
          rows_v.at[buf], agg_sh.at[dst_v.at[par, jj]], sems).wait()

    def group_body(gi, _):
      par = lax.rem(gi, 2)
      idx_wait(gi, par)

      @pl.when(gi + 1 < ngrp)
      def _():
        idx_copy(gi + 1, 1 - par)

      @pl.when(gi > 0)
      def _():
        drain_scatter(par, 0, 0)

      descs = [None] * GC
      descs[0] = pltpu.async_copy(
          h_hbm.at[src_v.at[par, 0]], rows_v.at[0], gsems[0])
      for jj in range(GC):
        descs[jj].wait()
        if jj + 1 < GC:
          if jj == 0:
            @pl.when(gi > 0)
            def _():
              drain_scatter(par, jj, (jj + 1) % 2)
          else:
            drain_scatter(par, jj, (jj + 1) % 2)
          descs[jj + 1] = pltpu.async_copy(
              h_hbm.at[src_v.at[par, jj + 1]], rows_v.at[(jj + 1) % 2],
              gsems[(jj + 1) % 2])
        scale_rows(par, jj, jj % 2)
        pltpu.async_copy(rows_v.at[jj % 2], agg_sh.at[dst_v.at[par, jj]],
                         sems, add=True)
      return 0
    lax.fori_loop(0, ngrp, group_body, 0)

    drain_scatter(lax.rem(ngrp - 1, 2), GC - 1, (GC - 1) % 2)
    drain_scatter(lax.rem(ngrp - 1, 2), GC - 2, (GC - 2) % 2)

    plsc.subcore_barrier()
    pltpu.sync_copy(agg_sh.at[pl.ds(sid * rpt, rpt)],
                    out_hbm.at[cid, pl.ds(sid * rpt, rpt)])

  return k


def _ln_body(h_ref, g_ref, b_ref, o_ref):
  x = h_ref[...]
  mu = jnp.mean(x, axis=1, keepdims=True)
  xc = x - mu
  var = jnp.mean(xc * xc, axis=1, keepdims=True)
  o_ref[...] = xc * lax.rsqrt(var + 1e-5) * g_ref[...] + b_ref[...]


@functools.lru_cache(maxsize=None)
def _make_ln(Npad, D, BLK=1024):
  grid = Npad // BLK
  return pl.pallas_call(
      _ln_body,
      grid=(grid,),
      in_specs=[
          pl.BlockSpec((BLK, D), lambda i: (i, 0)),
          pl.BlockSpec((1, D), lambda i: (0, 0)),
          pl.BlockSpec((1, D), lambda i: (0, 0)),
      ],
      out_specs=pl.BlockSpec((BLK, D), lambda i: (i, 0)),
      out_shape=jax.ShapeDtypeStruct((Npad, D), jnp.float32),
  )


def _conv_mlp_ln_body(hn_ref, p0_ref, p1_ref, W1_ref, b1_ref, W2_ref, b2_ref,
                      eps_ref, g_ref, bb_ref, o_ref):
  hn = hn_ref[...]
  t = (2.0 + eps_ref[0]) * hn + p0_ref[...] + p1_ref[...]
  t = jnp.maximum(t @ W1_ref[...] + b1_ref[...], 0.0) @ W2_ref[...] + b2_ref[...]
  t = jnp.maximum(t, 0.0)
  mu = jnp.mean(t, axis=1, keepdims=True)
  tc = t - mu
  var = jnp.mean(tc * tc, axis=1, keepdims=True)
  o_ref[...] = tc * lax.rsqrt(var + 1e-5) * g_ref[...] + bb_ref[...]


@functools.lru_cache(maxsize=None)
def _make_conv_mlp_ln(Npad, D, H, BLK=512):
  grid = Npad // BLK
  return pl.pallas_call(
      _conv_mlp_ln_body,
      grid=(grid,),
      in_specs=[
          pl.BlockSpec((BLK, D), lambda i: (i, 0)),
          pl.BlockSpec((BLK, D), lambda i: (i, 0)),
          pl.BlockSpec((BLK, D), lambda i: (i, 0)),
          pl.BlockSpec((D, H), lambda i: (0, 0)),
          pl.BlockSpec((1, H), lambda i: (0, 0)),
          pl.BlockSpec((H, H), lambda i: (0, 0)),
          pl.BlockSpec((1, H), lambda i: (0, 0)),
          pl.BlockSpec(memory_space=pltpu.SMEM),
          pl.BlockSpec((1, H), lambda i: (0, 0)),
          pl.BlockSpec((1, H), lambda i: (0, 0)),
      ],
      out_specs=pl.BlockSpec((BLK, H), lambda i: (i, 0)),
      out_shape=jax.ShapeDtypeStruct((Npad, H), jnp.float32),
  )


def _final_body(hn_ref, p0_ref, p1_ref, W1_ref, b1_ref, W2_ref, b2_ref,
                eps_ref, batch_ref, mW1_ref, mb1_ref, mW2_ref, mb2_ref,
                o_ref, sums_ref, cnt_ref, G, BLK):
  i = pl.program_id(0)

  @pl.when(i == 0)
  def _():
    sums_ref[...] = jnp.zeros_like(sums_ref)
    cnt_ref[...] = jnp.zeros_like(cnt_ref)

  hn = hn_ref[...]
  t = (2.0 + eps_ref[0]) * hn + p0_ref[...] + p1_ref[...]
  t = jnp.maximum(t @ W1_ref[...] + b1_ref[...], 0.0) @ W2_ref[...] + b2_ref[...]
  h2 = jnp.maximum(t, 0.0)

  b = batch_ref[0, 0]
  onehot = (b[None, :] == lax.broadcasted_iota(jnp.int32, (G, BLK), 0)
            ).astype(jnp.float32)
  sums_ref[...] += onehot @ h2
  cnt_ref[...] += jnp.broadcast_to(
      jnp.sum(onehot, axis=1, keepdims=True), cnt_ref.shape)

  @pl.when(i == pl.num_programs(0) - 1)
  def _():
    g = sums_ref[...] / jnp.maximum(cnt_ref[...], 1.0)
    o = jnp.maximum(g @ mW1_ref[...] + mb1_ref[...], 0.0) @ mW2_ref[...] \
        + mb2_ref[...]
    o_ref[...] = o


@functools.lru_cache(maxsize=None)
def _make_final(Npad, H, G, BLK=512):
  grid = Npad // BLK
  P = 128
  return pl.pallas_call(
      functools.partial(_final_body, G=G, BLK=BLK),
      grid=(grid,),
      in_specs=[
          pl.BlockSpec((BLK, H), lambda i: (i, 0)),
          pl.BlockSpec((BLK, H), lambda i: (i, 0)),
          pl.BlockSpec((BLK, H), lambda i: (i, 0)),
          pl.BlockSpec((H, H), lambda i: (0, 0)),
          pl.BlockSpec((1, H), lambda i: (0, 0)),
          pl.BlockSpec((H, H), lambda i: (0, 0)),
          pl.BlockSpec((1, H), lambda i: (0, 0)),
          pl.BlockSpec(memory_space=pltpu.SMEM),
          pl.BlockSpec((1, 1, BLK), lambda i: (i, 0, 0)),
          pl.BlockSpec((H, P), lambda i: (0, 0)),
          pl.BlockSpec((1, P), lambda i: (0, 0)),
          pl.BlockSpec((P, P), lambda i: (0, 0)),
          pl.BlockSpec((1, P), lambda i: (0, 0)),
      ],
      out_specs=pl.BlockSpec((G, P), lambda i: (0, 0)),
      out_shape=jax.ShapeDtypeStruct((G, P), jnp.float32),
      scratch_shapes=[
          pltpu.VMEM((G, H), jnp.float32),
          pltpu.VMEM((G, H), jnp.float32),
      ],
  )


def kernel(x, edge_index, edge_attr, batch, emb, ln1_g, ln1_b, W1a, b1a, W2a,
           b2a, eps1, ln2_g, ln2_b, W1b, b1b, W2b, b2b, eps2, mW1, mb1, mW2,
           mb2):
  N = x.shape[0]
  E = edge_index.shape[1]
  V, D = emb.shape
  H = W1a.shape[1]
  G = 64
  OP = mW2.shape[1]
  HH = mW1.shape[1]
  KD = D // LANES
  C = 128

  GC = 5
  Npad = ((N + 8 * NW - 1) // (8 * NW)) * (8 * NW)
  gsz = NS * GC * C
  G0, G1 = 29, 3
  ET = (G0 + G1) * gsz
  assert ET >= E

  x_pad = jnp.zeros((Npad,), jnp.int32).at[:N].set(x.astype(jnp.int32))
  x_pad = x_pad.reshape(NW, -1, 80)

  def _split(a, fill):
    ap = jnp.full((ET,), fill, a.dtype).at[:E].set(a)
    gmax = max(G0, G1)
    p0 = ap[:G0 * gsz].reshape(NS, G0, GC, C)
    p1 = ap[G0 * gsz:].reshape(NS, G1, GC, C)
    p0 = jnp.pad(p0, ((0, 0), (0, gmax - G0), (0, 0), (0, 0)))
    p1 = jnp.pad(p1, ((0, 0), (0, gmax - G1), (0, 0), (0, 0)))
    return jnp.stack([p0, p1])

  src = _split(edge_index[0].astype(jnp.int32), 0)
  dst = _split(edge_index[1].astype(jnp.int32), 0)
  w = _split(edge_attr, 0.0)

  batch_pad = jnp.full((Npad,), G, jnp.int32).at[:N].set(batch.astype(jnp.int32))
  batch_pad = batch_pad.reshape(-1, 1, 512)

  r1 = lambda a: a.reshape(1, -1)

  h = _make_emb_gather(V, D, Npad)(emb, x_pad)
  h1n = _make_ln(Npad, D)(h, r1(ln1_g), r1(ln1_b))

  agg_fn = _make_edge_agg(Npad, D, G0, G1, C)
  parts1 = agg_fn(h1n, src, dst, w)

  h2n = _make_conv_mlp_ln(Npad, D, H)(
      h1n, parts1[0], parts1[1], W1a, r1(b1a), W2a, r1(b2a),
      eps1.reshape(1), r1(ln2_g), r1(ln2_b))

  parts2 = agg_fn(h2n, src, dst, w)

  P = 128
  mW1p = jnp.zeros((H, P), jnp.float32).at[:, :HH].set(mW1)
  mb1p = jnp.zeros((P,), jnp.float32).at[:HH].set(mb1)
  mW2p = jnp.zeros((P, P), jnp.float32).at[:HH, :OP].set(mW2)
  mb2p = jnp.zeros((P,), jnp.float32).at[:OP].set(mb2)

  out = _make_final(Npad, H, G)(
      h2n, parts2[0], parts2[1], W1b, r1(b1b), W2b, r1(b2b),
      eps2.reshape(1), batch_pad, mW1p, r1(mb1p), mW2p, r1(mb2p))
  return out[:, :OP]
